# baseline (speedup 1.0000x reference)
import functools

import jax
import jax.numpy as jnp
from jax import lax
from jax.experimental import pallas as pl
from jax.experimental.pallas import tpu as pltpu
from jax.experimental.pallas import tpu_sc as plsc

NC = 2
NS = 16
NW = NC * NS

D_MODEL = 768
N_TOK = 2048
N_EXP = 16
K = 2
H_DIM = 2 * D_MODEL
SCALE = 0.1
T_BLK = 288
C_PAIR = 128
N_PAIR = N_TOK * K
N_CHUNK = N_PAIR // C_PAIR
N_BLK = -(-N_PAIR // T_BLK) + N_EXP
S_ROWS = N_BLK * T_BLK


def _router_meta_kernel(x_ref, wg_ref, bg_ref,
                        dest_ref, gbc_ref, be_ref, valid_ref,
                        ent_ref, p_ref, c_ref):
    xx = x_ref[...]
    nrm2 = jnp.sum(xx * xx, axis=1, keepdims=True)
    nrm = jnp.sqrt(nrm2)
    mu = jnp.mean(nrm)
    sd = jnp.sqrt(jnp.sum((nrm - mu) ** 2) / (N_TOK - 1))
    logits = jax.lax.dot_general(
        xx, wg_ref[...], (((1,), (1,)), ((), ())),
        preferred_element_type=jnp.float32)
    logits = logits + bg_ref[...][None, :] + SCALE * (nrm - mu) / (sd + 1e-6)

    lane = jax.lax.broadcasted_iota(jnp.int32, (N_TOK, N_EXP), 1)
    m1 = jnp.max(logits, axis=1, keepdims=True)
    a1 = jnp.min(jnp.where(logits == m1, lane, N_EXP), axis=1, keepdims=True)
    masked = jnp.where(lane == a1, -jnp.inf, logits)
    m2 = jnp.max(masked, axis=1, keepdims=True)
    a2 = jnp.min(jnp.where(masked == m2, lane, N_EXP), axis=1, keepdims=True)

    e2 = jnp.exp(m2 - m1)
    g1 = 1.0 / (1.0 + e2)
    g2 = e2 / (1.0 + e2)
    ent = -(g1 * jnp.log(jnp.maximum(g1, 1e-8)) +
            g2 * jnp.log(jnp.maximum(g2, 1e-8)))
    ent_ref[...] = jnp.sum(ent, axis=0, keepdims=True) / N_TOK
    gbc_ref[...] = jnp.concatenate(
        [jnp.broadcast_to(g1, (N_TOK, 16)),
         jnp.broadcast_to(g2, (N_TOK, 16))], axis=0)

    e_pair = jnp.concatenate([a1, a2], axis=0)
    lane_p = jax.lax.broadcasted_iota(jnp.int32, (N_PAIR, N_EXP), 1)
    p_ref[...] = (lane_p == e_pair).astype(jnp.float32)

    CH = 1024
    tri = (jax.lax.broadcasted_iota(jnp.int32, (CH, CH), 0) >=
           jax.lax.broadcasted_iota(jnp.int32, (CH, CH), 1)
           ).astype(jnp.bfloat16)

    def body(b, carry):
        pb = p_ref[pl.ds(b * CH, CH), :]
        c_ref[pl.ds(b * CH, CH), :] = carry + jax.lax.dot_general(
            tri, pb.astype(jnp.bfloat16), (((1,), (0,)), ((), ())),
            preferred_element_type=jnp.float32)
        return carry + jnp.sum(pb, axis=0, keepdims=True)

    counts = jax.lax.fori_loop(0, N_PAIR // CH, body,
                               jnp.zeros((1, N_EXP), jnp.float32))
    nblk = jnp.floor((counts + (T_BLK - 1)) / T_BLK)
    tri16 = (jax.lax.broadcasted_iota(jnp.int32, (N_EXP, N_EXP), 0) <
             jax.lax.broadcasted_iota(jnp.int32, (N_EXP, N_EXP), 1)
             ).astype(jnp.float32)
    blk_start = jax.lax.dot_general(
        nblk, tri16, (((1,), (0,)), ((), ())),
        preferred_element_type=jnp.float32)
    slot_base = blk_start * T_BLK

    pp = p_ref[...]
    d_full = c_ref[...] * pp + pp * slot_base
    dest_ref[...] = (jnp.sum(
        d_full.reshape(N_CHUNK, C_PAIR, N_EXP), axis=2) - 1.0
        ).astype(jnp.int32)

    blk_end = blk_start + nblk
    bi = jax.lax.broadcasted_iota(jnp.int32, (N_BLK, N_EXP), 0
                                  ).astype(jnp.float32)
    be = jnp.sum((blk_end <= bi).astype(jnp.int32), axis=1, keepdims=True)
    be_ref[...] = jnp.minimum(be, N_EXP - 1)
    tot = jnp.sum(nblk, axis=1, keepdims=True)
    valid_ref[...] = (bi[:, 0:1] < tot).astype(jnp.int32)


def _router_meta(x_flat, wg, bg):
    return pl.pallas_call(
        _router_meta_kernel,
        out_shape=(
            jax.ShapeDtypeStruct((N_CHUNK, C_PAIR), jnp.int32),
            jax.ShapeDtypeStruct((N_PAIR, 16), jnp.float32),
            jax.ShapeDtypeStruct((N_BLK, 1), jnp.int32),
            jax.ShapeDtypeStruct((N_BLK, 1), jnp.int32),
            jax.ShapeDtypeStruct((1, 1), jnp.float32),
        ),
        scratch_shapes=[
            pltpu.VMEM((N_PAIR, N_EXP), jnp.float32),
            pltpu.VMEM((N_PAIR, N_EXP), jnp.float32),
        ],
    )(x_flat, wg, bg)


HS = 1
H_SUB = H_DIM // HS


def _ffn_kernel(be_ref, valid_ref, xd_ref, w1_ref, b1_ref, w2_ref, b2_ref,
                yd_ref):
    b = pl.program_id(0)
    hh = pl.program_id(1)

    @pl.when(valid_ref[b] != 0)
    def _():
        xb = xd_ref[0]
        h = jax.lax.dot_general(
            xb, w1_ref[0], (((1,), (1,)), ((), ())),
            preferred_element_type=jnp.float32)
        h = jnp.maximum(h + b1_ref[0], 0.0)
        y = jax.lax.dot_general(
            h, w2_ref[0], (((1,), (1,)), ((), ())),
            preferred_element_type=jnp.float32)

        @pl.when(hh == 0)
        def _():
            yd_ref[0] = y + b2_ref[0]

        @pl.when(hh != 0)
        def _():
            yd_ref[0] += y


def _ffn(xd3, w1, b1, w2, b2, block_expert, block_valid):
    grid_spec = pltpu.PrefetchScalarGridSpec(
        num_scalar_prefetch=2,
        grid=(N_BLK, HS),
        in_specs=[
            pl.BlockSpec((1, T_BLK, D_MODEL),
                         lambda b, h, be, v: (v[b] * b, 0, 0)),
            pl.BlockSpec((1, H_SUB, D_MODEL),
                         lambda b, h, be, v: (be[b] * HS + h, 0, 0)),
            pl.BlockSpec((1, 1, H_SUB), lambda b, h, be, v: (be[b], 0, h)),
            pl.BlockSpec((1, D_MODEL, H_SUB),
                         lambda b, h, be, v: (be[b], 0, h)),
            pl.BlockSpec((1, 1, D_MODEL), lambda b, h, be, v: (be[b], 0, 0)),
        ],
        out_specs=pl.BlockSpec((1, T_BLK, D_MODEL),
                               lambda b, h, be, v: (b, 0, 0)),
    )
    return pl.pallas_call(
        _ffn_kernel,
        grid_spec=grid_spec,
        out_shape=jax.ShapeDtypeStruct((N_BLK, T_BLK, D_MODEL), jnp.float32),
    )(block_expert, block_valid, xd3,
      w1.reshape(N_EXP * HS, H_SUB, D_MODEL), b1.reshape(N_EXP, 1, H_DIM),
      w2, b2.reshape(N_EXP, 1, D_MODEL))


_SC_MESH = plsc.VectorSubcoreMesh(core_axis_name="c", subcore_axis_name="s")


@functools.partial(
    pl.kernel,
    out_type=jax.ShapeDtypeStruct((S_ROWS, D_MODEL), jnp.float32),
    mesh=_SC_MESH,
    scratch_types=[
        pltpu.VMEM((C_PAIR,), jnp.int32),
        pltpu.VMEM((C_PAIR, D_MODEL), jnp.float32),
        pltpu.SemaphoreType.DMA,
    ],
)
def _dispatch(x_hbm, dest_hbm, xd_hbm, idx_v, rows_v, sem):
    wid = lax.axis_index("s") * NC + lax.axis_index("c")
    tbase = (wid % NS) * C_PAIR
    pltpu.sync_copy(dest_hbm.at[wid], idx_v)
    pltpu.sync_copy(x_hbm.at[pl.ds(tbase, C_PAIR)], rows_v)
    pltpu.async_copy(rows_v, xd_hbm.at[idx_v], sem).wait()


_TOK_W = N_TOK // NW


@functools.partial(
    pl.kernel,
    out_type=jax.ShapeDtypeStruct((N_TOK, D_MODEL), jnp.float32),
    mesh=_SC_MESH,
    scratch_types=[
        pltpu.VMEM((_TOK_W // 2,), jnp.int32),
        pltpu.VMEM((_TOK_W // 2,), jnp.int32),
        pltpu.VMEM((_TOK_W // 2,), jnp.int32),
        pltpu.VMEM((_TOK_W // 2,), jnp.int32),
        pltpu.VMEM((_TOK_W // 2, D_MODEL), jnp.float32),
        pltpu.VMEM((_TOK_W // 2, D_MODEL), jnp.float32),
        pltpu.VMEM((_TOK_W // 2, D_MODEL), jnp.float32),
        pltpu.VMEM((_TOK_W // 2, D_MODEL), jnp.float32),
        pltpu.VMEM((_TOK_W, 16), jnp.float32),
        pltpu.VMEM((_TOK_W, 16), jnp.float32),
        pltpu.SemaphoreType.DMA,
        pltpu.SemaphoreType.DMA,
        pltpu.SemaphoreType.DMA,
    ],
)
def _combine(yd_hbm, dest_hbm, gbc_hbm, out_hbm,
             i0a_v, i1a_v, i0b_v, i1b_v, r0a_v, r1a_v, r0b_v, r1b_v,
             g0_v, g1_v, sema, semb, semw):
    HW = _TOK_W // 2
    wid = lax.axis_index("s") * NC + lax.axis_index("c")
    t0 = wid * _TOK_W
    row0, col0 = wid // 2, (wid % 2) * _TOK_W
    pltpu.sync_copy(dest_hbm.at[row0, pl.ds(col0, HW)], i0a_v)
    pltpu.sync_copy(dest_hbm.at[NS + row0, pl.ds(col0, HW)], i1a_v)
    pltpu.sync_copy(dest_hbm.at[row0, pl.ds(col0 + HW, HW)], i0b_v)
    pltpu.sync_copy(dest_hbm.at[NS + row0, pl.ds(col0 + HW, HW)], i1b_v)
    ca0 = pltpu.async_copy(yd_hbm.at[i0a_v], r0a_v, sema)
    ca1 = pltpu.async_copy(yd_hbm.at[i1a_v], r1a_v, sema)
    cb0 = pltpu.async_copy(yd_hbm.at[i0b_v], r0b_v, semb)
    cb1 = pltpu.async_copy(yd_hbm.at[i1b_v], r1b_v, semb)
    pltpu.sync_copy(gbc_hbm.at[pl.ds(t0, _TOK_W)], g0_v)
    pltpu.sync_copy(gbc_hbm.at[pl.ds(N_TOK + t0, _TOK_W)], g1_v)

    def half_body(r0, r1, goff):
        def body(j, _):
            ga = g0_v[goff + j]
            gb = g1_v[goff + j]
            for c in range(D_MODEL // 16):
                sl = pl.ds(c * 16, 16)
                r0[j, sl] = r0[j, sl] * ga + r1[j, sl] * gb
            return 0
        lax.fori_loop(0, HW, body, 0)

    ca0.wait()
    ca1.wait()
    half_body(r0a_v, r1a_v, 0)
    wa = pltpu.async_copy(r0a_v, out_hbm.at[pl.ds(t0, HW)], semw)
    cb0.wait()
    cb1.wait()
    half_body(r0b_v, r1b_v, HW)
    wa.wait()
    pltpu.sync_copy(r0b_v, out_hbm.at[pl.ds(t0 + HW, HW)])


def kernel(x, Wg, bg, W1, b1, W2, b2):
    B, N, D = x.shape
    x_flat = x.reshape(N, D)
    dest, gbc, block_expert, block_valid, ent = _router_meta(
        x_flat, Wg, bg)
    block_expert = block_expert.reshape(N_BLK)
    block_valid = block_valid.reshape(N_BLK)

    xd = _dispatch(x_flat, dest)
    yd3 = _ffn(xd.reshape(N_BLK, T_BLK, D), W1, b1, W2, b2,
               block_expert, block_valid)
    out_flat = _combine(yd3.reshape(S_ROWS, D), dest, gbc)
    return (out_flat.reshape(B, N, D), ent[0, 0])


# speedup vs baseline: 1.0331x; 1.0331x over previous
import functools

import jax
import jax.numpy as jnp
from jax import lax
from jax.experimental import pallas as pl
from jax.experimental.pallas import tpu as pltpu
from jax.experimental.pallas import tpu_sc as plsc

NC = 2
NS = 16
NW = NC * NS

D_MODEL = 768
N_TOK = 2048
N_EXP = 16
K = 2
H_DIM = 2 * D_MODEL
SCALE = 0.1
T_BLK = 320
C_PAIR = 128
N_PAIR = N_TOK * K
N_CHUNK = N_PAIR // C_PAIR
N_BLK = -(-N_PAIR // T_BLK) + N_EXP
S_ROWS = N_BLK * T_BLK


def _router_meta_kernel(x_ref, wg_ref, bg_ref,
                        dest_ref, gbc_ref, be_ref, valid_ref,
                        ent_ref, p_ref, c_ref):
    xx = x_ref[...]
    nrm2 = jnp.sum(xx * xx, axis=1, keepdims=True)
    nrm = jnp.sqrt(nrm2)
    mu = jnp.mean(nrm)
    sd = jnp.sqrt(jnp.sum((nrm - mu) ** 2) / (N_TOK - 1))
    logits = jax.lax.dot_general(
        xx, wg_ref[...], (((1,), (1,)), ((), ())),
        preferred_element_type=jnp.float32)
    logits = logits + bg_ref[...][None, :] + SCALE * (nrm - mu) / (sd + 1e-6)

    lane = jax.lax.broadcasted_iota(jnp.int32, (N_TOK, N_EXP), 1)
    m1 = jnp.max(logits, axis=1, keepdims=True)
    a1 = jnp.min(jnp.where(logits == m1, lane, N_EXP), axis=1, keepdims=True)
    masked = jnp.where(lane == a1, -jnp.inf, logits)
    m2 = jnp.max(masked, axis=1, keepdims=True)
    a2 = jnp.min(jnp.where(masked == m2, lane, N_EXP), axis=1, keepdims=True)

    e2 = jnp.exp(m2 - m1)
    g1 = 1.0 / (1.0 + e2)
    g2 = e2 / (1.0 + e2)
    ent = -(g1 * jnp.log(jnp.maximum(g1, 1e-8)) +
            g2 * jnp.log(jnp.maximum(g2, 1e-8)))
    ent_ref[...] = jnp.sum(ent, axis=0, keepdims=True) / N_TOK
    gbc_ref[...] = jnp.concatenate(
        [jnp.broadcast_to(g1, (N_TOK, 16)),
         jnp.broadcast_to(g2, (N_TOK, 16))], axis=0)

    e_pair = jnp.concatenate([a1, a2], axis=0)
    lane_p = jax.lax.broadcasted_iota(jnp.int32, (N_PAIR, N_EXP), 1)
    p_ref[...] = (lane_p == e_pair).astype(jnp.float32)

    CH = 1024
    tri = (jax.lax.broadcasted_iota(jnp.int32, (CH, CH), 0) >=
           jax.lax.broadcasted_iota(jnp.int32, (CH, CH), 1)
           ).astype(jnp.bfloat16)

    def body(b, carry):
        pb = p_ref[pl.ds(b * CH, CH), :]
        c_ref[pl.ds(b * CH, CH), :] = carry + jax.lax.dot_general(
            tri, pb.astype(jnp.bfloat16), (((1,), (0,)), ((), ())),
            preferred_element_type=jnp.float32)
        return carry + jnp.sum(pb, axis=0, keepdims=True)

    counts = jax.lax.fori_loop(0, N_PAIR // CH, body,
                               jnp.zeros((1, N_EXP), jnp.float32))
    nblk = jnp.floor((counts + (T_BLK - 1)) / T_BLK)
    tri16 = (jax.lax.broadcasted_iota(jnp.int32, (N_EXP, N_EXP), 0) <
             jax.lax.broadcasted_iota(jnp.int32, (N_EXP, N_EXP), 1)
             ).astype(jnp.float32)
    blk_start = jax.lax.dot_general(
        nblk, tri16, (((1,), (0,)), ((), ())),
        preferred_element_type=jnp.float32)
    slot_base = blk_start * T_BLK

    pp = p_ref[...]
    d_full = c_ref[...] * pp + pp * slot_base
    dest_ref[...] = (jnp.sum(
        d_full.reshape(N_CHUNK, C_PAIR, N_EXP), axis=2) - 1.0
        ).astype(jnp.int32)

    blk_end = blk_start + nblk
    bi = jax.lax.broadcasted_iota(jnp.int32, (N_BLK, N_EXP), 0
                                  ).astype(jnp.float32)
    be = jnp.sum((blk_end <= bi).astype(jnp.int32), axis=1, keepdims=True)
    be_ref[...] = jnp.minimum(be, N_EXP - 1)
    tot = jnp.sum(nblk, axis=1, keepdims=True)
    valid_ref[...] = (bi[:, 0:1] < tot).astype(jnp.int32)


def _router_meta(x_flat, wg, bg):
    return pl.pallas_call(
        _router_meta_kernel,
        out_shape=(
            jax.ShapeDtypeStruct((N_CHUNK, C_PAIR), jnp.int32),
            jax.ShapeDtypeStruct((N_PAIR, 16), jnp.float32),
            jax.ShapeDtypeStruct((N_BLK, 1), jnp.int32),
            jax.ShapeDtypeStruct((N_BLK, 1), jnp.int32),
            jax.ShapeDtypeStruct((1, 1), jnp.float32),
        ),
        scratch_shapes=[
            pltpu.VMEM((N_PAIR, N_EXP), jnp.float32),
            pltpu.VMEM((N_PAIR, N_EXP), jnp.float32),
        ],
    )(x_flat, wg, bg)


HS = 1
H_SUB = H_DIM // HS


def _ffn_kernel(be_ref, valid_ref, xd_ref, w1_ref, b1_ref, w2_ref, b2_ref,
                yd_ref):
    b = pl.program_id(0)
    hh = pl.program_id(1)

    @pl.when(valid_ref[b] != 0)
    def _():
        xb = xd_ref[0]
        h = jax.lax.dot_general(
            xb, w1_ref[0], (((1,), (1,)), ((), ())),
            preferred_element_type=jnp.float32)
        h = jnp.maximum(h + b1_ref[0], 0.0)
        y = jax.lax.dot_general(
            h, w2_ref[0], (((1,), (1,)), ((), ())),
            preferred_element_type=jnp.float32)

        @pl.when(hh == 0)
        def _():
            yd_ref[0] = y + b2_ref[0]

        @pl.when(hh != 0)
        def _():
            yd_ref[0] += y


def _ffn(xd3, w1, b1, w2, b2, block_expert, block_valid):
    grid_spec = pltpu.PrefetchScalarGridSpec(
        num_scalar_prefetch=2,
        grid=(N_BLK, HS),
        in_specs=[
            pl.BlockSpec((1, T_BLK, D_MODEL),
                         lambda b, h, be, v: (v[b] * b, 0, 0)),
            pl.BlockSpec((1, H_SUB, D_MODEL),
                         lambda b, h, be, v: (be[b] * HS + h, 0, 0)),
            pl.BlockSpec((1, 1, H_SUB), lambda b, h, be, v: (be[b], 0, h)),
            pl.BlockSpec((1, D_MODEL, H_SUB),
                         lambda b, h, be, v: (be[b], 0, h)),
            pl.BlockSpec((1, 1, D_MODEL), lambda b, h, be, v: (be[b], 0, 0)),
        ],
        out_specs=pl.BlockSpec((1, T_BLK, D_MODEL),
                               lambda b, h, be, v: (b, 0, 0)),
    )
    return pl.pallas_call(
        _ffn_kernel,
        grid_spec=grid_spec,
        out_shape=jax.ShapeDtypeStruct((N_BLK, T_BLK, D_MODEL), jnp.float32),
    )(block_expert, block_valid, xd3,
      w1.reshape(N_EXP * HS, H_SUB, D_MODEL), b1.reshape(N_EXP, 1, H_DIM),
      w2, b2.reshape(N_EXP, 1, D_MODEL))


_SC_MESH = plsc.VectorSubcoreMesh(core_axis_name="c", subcore_axis_name="s")


@functools.partial(
    pl.kernel,
    out_type=jax.ShapeDtypeStruct((S_ROWS, D_MODEL), jnp.float32),
    mesh=_SC_MESH,
    scratch_types=[
        pltpu.VMEM((C_PAIR,), jnp.int32),
        pltpu.VMEM((C_PAIR, D_MODEL), jnp.float32),
        pltpu.SemaphoreType.DMA,
    ],
)
def _dispatch(x_hbm, dest_hbm, xd_hbm, idx_v, rows_v, sem):
    wid = lax.axis_index("s") * NC + lax.axis_index("c")
    tbase = (wid % NS) * C_PAIR
    pltpu.sync_copy(dest_hbm.at[wid], idx_v)
    pltpu.sync_copy(x_hbm.at[pl.ds(tbase, C_PAIR)], rows_v)
    pltpu.async_copy(rows_v, xd_hbm.at[idx_v], sem).wait()


_TOK_W = N_TOK // NW


@functools.partial(
    pl.kernel,
    out_type=jax.ShapeDtypeStruct((N_TOK, D_MODEL), jnp.float32),
    mesh=_SC_MESH,
    scratch_types=[
        pltpu.VMEM((_TOK_W // 2,), jnp.int32),
        pltpu.VMEM((_TOK_W // 2,), jnp.int32),
        pltpu.VMEM((_TOK_W // 2,), jnp.int32),
        pltpu.VMEM((_TOK_W // 2,), jnp.int32),
        pltpu.VMEM((_TOK_W // 2, D_MODEL), jnp.float32),
        pltpu.VMEM((_TOK_W // 2, D_MODEL), jnp.float32),
        pltpu.VMEM((_TOK_W // 2, D_MODEL), jnp.float32),
        pltpu.VMEM((_TOK_W // 2, D_MODEL), jnp.float32),
        pltpu.VMEM((_TOK_W, 16), jnp.float32),
        pltpu.VMEM((_TOK_W, 16), jnp.float32),
        pltpu.SemaphoreType.DMA,
        pltpu.SemaphoreType.DMA,
        pltpu.SemaphoreType.DMA,
    ],
)
def _combine(yd_hbm, dest_hbm, gbc_hbm, out_hbm,
             i0a_v, i1a_v, i0b_v, i1b_v, r0a_v, r1a_v, r0b_v, r1b_v,
             g0_v, g1_v, sema, semb, semw):
    HW = _TOK_W // 2
    wid = lax.axis_index("s") * NC + lax.axis_index("c")
    t0 = wid * _TOK_W
    row0, col0 = wid // 2, (wid % 2) * _TOK_W
    pltpu.sync_copy(dest_hbm.at[row0, pl.ds(col0, HW)], i0a_v)
    pltpu.sync_copy(dest_hbm.at[NS + row0, pl.ds(col0, HW)], i1a_v)
    pltpu.sync_copy(dest_hbm.at[row0, pl.ds(col0 + HW, HW)], i0b_v)
    pltpu.sync_copy(dest_hbm.at[NS + row0, pl.ds(col0 + HW, HW)], i1b_v)
    ca0 = pltpu.async_copy(yd_hbm.at[i0a_v], r0a_v, sema)
    ca1 = pltpu.async_copy(yd_hbm.at[i1a_v], r1a_v, sema)
    cb0 = pltpu.async_copy(yd_hbm.at[i0b_v], r0b_v, semb)
    cb1 = pltpu.async_copy(yd_hbm.at[i1b_v], r1b_v, semb)
    pltpu.sync_copy(gbc_hbm.at[pl.ds(t0, _TOK_W)], g0_v)
    pltpu.sync_copy(gbc_hbm.at[pl.ds(N_TOK + t0, _TOK_W)], g1_v)

    def half_body(r0, r1, goff):
        def body(j, _):
            ga = g0_v[goff + j]
            gb = g1_v[goff + j]
            for c in range(D_MODEL // 16):
                sl = pl.ds(c * 16, 16)
                r0[j, sl] = r0[j, sl] * ga + r1[j, sl] * gb
            return 0
        lax.fori_loop(0, HW, body, 0)

    ca0.wait()
    ca1.wait()
    half_body(r0a_v, r1a_v, 0)
    wa = pltpu.async_copy(r0a_v, out_hbm.at[pl.ds(t0, HW)], semw)
    cb0.wait()
    cb1.wait()
    half_body(r0b_v, r1b_v, HW)
    wa.wait()
    pltpu.sync_copy(r0b_v, out_hbm.at[pl.ds(t0 + HW, HW)])


def kernel(x, Wg, bg, W1, b1, W2, b2):
    B, N, D = x.shape
    x_flat = x.reshape(N, D)
    dest, gbc, block_expert, block_valid, ent = _router_meta(
        x_flat, Wg, bg)
    block_expert = block_expert.reshape(N_BLK)
    block_valid = block_valid.reshape(N_BLK)

    xd = _dispatch(x_flat, dest)
    yd3 = _ffn(xd.reshape(N_BLK, T_BLK, D), W1, b1, W2, b2,
               block_expert, block_valid)
    out_flat = _combine(yd3.reshape(S_ROWS, D), dest, gbc)
    return (out_flat.reshape(B, N, D), ent[0, 0])
